# baseline (speedup 1.0000x reference)
import functools

import jax
import jax.numpy as jnp
from jax import lax
from jax.experimental import pallas as pl
from jax.experimental.pallas import tpu as pltpu
from jax.experimental.pallas import tpu_sc as plsc

_M = 4
_K = 8192
_D = 256
_SPLIT = _D // _M
_N = 8192
_TN = 1024

_NW = 32
_ROWS_W = (_M * _N) // _NW
_BLK = 128


def _main_body(x_ref, r_ref, c_ref, idx_ref, soft_ref, stats_ref):
    xt = x_ref[...]
    rm = r_ref[0]
    split = jnp.dot(xt, rm, preferred_element_type=jnp.float32)
    c = c_ref[0]
    cc = jnp.sum(c * c, axis=1)
    dots2 = jax.lax.dot_general(2.0 * split, c, (((1,), (1,)), ((), ())),
                                preferred_element_type=jnp.float32)
    s = dots2 - cc[None, :]
    mx = jnp.max(s, axis=1, keepdims=True)
    ge = s >= mx
    iota = jax.lax.broadcasted_iota(jnp.int32, (_TN, _K), 1)
    idx = jnp.min(jnp.where(ge, iota, _K), axis=1, keepdims=True)
    p = jnp.exp(s - mx).astype(jnp.bfloat16)
    lane65 = jax.lax.broadcasted_iota(jnp.int32, (_K, _SPLIT + 1), 1)
    c_ext = jnp.where(lane65 < _SPLIT, jnp.pad(c, ((0, 0), (0, 1))), 1.0)
    soft_den = jax.lax.dot_general(p, c_ext, (((1,), (0,)), ((), ())),
                                   preferred_element_type=jnp.float32)
    den = soft_den[:, _SPLIT:_SPLIT + 1]
    soft = soft_den[:, :_SPLIT] / den
    xx = jnp.sum(split * split, axis=1, keepdims=True)
    d1 = split - soft
    lane = jax.lax.broadcasted_iota(jnp.int32, (1, 128), 1)
    v = jnp.where(lane == 0, jnp.sum(d1 * d1),
                  jnp.where(lane == 1, jnp.sum(xx - mx), 0.0))
    stats_ref[...] = v.reshape(1, 1, 1, 128)
    soft_ref[...] = soft.reshape(1, _TN, _SPLIT)
    idx_ref[...] = idx.reshape(1, _TN, 1)


def _reg_body(r_ref, o_ref):
    r = r_ref[...]
    rrt = jax.lax.dot_general(r, r, (((1,), (1,)), ((), ())),
                              preferred_element_type=jnp.float32)
    i0 = jax.lax.broadcasted_iota(jnp.int32, (_D, _D), 0)
    i1 = jax.lax.broadcasted_iota(jnp.int32, (_D, _D), 1)
    d = rrt - (i0 == i1).astype(jnp.float32)
    o_ref[...] = (jnp.sum(d * d) / float(_D * _D)).reshape(1, 1)


def _joint_body(cs_hbm, idx_hbm, soft_hbm, out_hbm, idx_v, rows_v, soft_v,
                acc_v, sem):
    wid = lax.axis_index("s") * 2 + lax.axis_index("c")
    moff = (wid // (_N // _ROWS_W)) * _K

    for g in range(4):
        acc_v[pl.ds(g * 16, 16)] = jnp.zeros((16,), jnp.float32)

    def blk_body(b, _):
        base = wid * _ROWS_W + b * _BLK
        pltpu.sync_copy(idx_hbm.at[pl.ds(base, _BLK)], idx_v)
        for j in range(_BLK // 16):
            sl = pl.ds(j * 16, 16)
            idx_v[sl] = idx_v[sl] + moff
        pltpu.async_copy(cs_hbm.at[idx_v], rows_v, sem).wait()
        pltpu.sync_copy(soft_hbm.at[pl.ds(base, _BLK)], soft_v)

        def row_body(r, _):
            for g in range(4):
                sl = pl.ds(g * 16, 16)
                d = soft_v[r, sl] - rows_v[r, sl]
                acc_v[pl.ds(g * 16, 16)] = acc_v[pl.ds(g * 16, 16)] + d * d
            return 0

        lax.fori_loop(0, _BLK, row_body, 0)
        return 0

    lax.fori_loop(0, _ROWS_W // _BLK, blk_body, 0)
    tot = acc_v[pl.ds(0, 16)] + acc_v[pl.ds(16, 16)]
    tot = tot + acc_v[pl.ds(32, 16)] + acc_v[pl.ds(48, 16)]
    acc_v[pl.ds(0, 16)] = tot
    pltpu.sync_copy(acc_v.at[pl.ds(0, 16)], out_hbm.at[pl.ds(wid * 16, 16)])


def kernel(x, codebook0, codebook1, codebook2, codebook3, rotateMatrix):
    cs = jnp.stack([codebook0, codebook1, codebook2, codebook3], axis=0)
    rs = rotateMatrix.reshape(_D, _M, _SPLIT).transpose(1, 0, 2)
    nt = _N // _TN
    idx_out, soft, stats = pl.pallas_call(
        _main_body,
        grid=(_M, nt),
        in_specs=[
            pl.BlockSpec((_TN, _D), lambda m, n: (n, 0)),
            pl.BlockSpec((1, _D, _SPLIT), lambda m, n: (m, 0, 0)),
            pl.BlockSpec((1, _K, _SPLIT), lambda m, n: (m, 0, 0)),
        ],
        out_specs=[
            pl.BlockSpec((1, _TN, 1), lambda m, n: (m, n, 0)),
            pl.BlockSpec((1, _TN, _SPLIT), lambda m, n: (m, n, 0)),
            pl.BlockSpec((1, 1, 1, 128), lambda m, n: (m, n, 0, 0)),
        ],
        out_shape=[
            jax.ShapeDtypeStruct((_M, _N, 1), jnp.int32),
            jax.ShapeDtypeStruct((_M, _N, _SPLIT), jnp.float32),
            jax.ShapeDtypeStruct((_M, nt, 1, 128), jnp.float32),
        ],
    )(x, rs, cs)
    reg = pl.pallas_call(
        _reg_body,
        out_shape=jax.ShapeDtypeStruct((1, 1), jnp.float32),
    )(rotateMatrix)

    cs_pad = jnp.pad(cs.reshape(_M * _K, _SPLIT), ((0, 0), (0, 128 - _SPLIT)))
    joint_parts = pl.kernel(
        _joint_body,
        out_type=jax.ShapeDtypeStruct((_NW * 16,), jnp.float32),
        mesh=plsc.VectorSubcoreMesh(core_axis_name="c", subcore_axis_name="s"),
        scratch_types=[
            pltpu.VMEM((_BLK,), jnp.int32),
            pltpu.VMEM((_BLK, 128), jnp.float32),
            pltpu.VMEM((_BLK, _SPLIT), jnp.float32),
            pltpu.VMEM((64,), jnp.float32),
            pltpu.SemaphoreType.DMA,
        ],
    )(cs_pad, idx_out.reshape(_M * _N), soft.reshape(_M * _N, _SPLIT))

    hardCodes = idx_out.reshape(_M, _N).T
    st = jnp.sum(stats.reshape(_M * nt, 128), axis=0)
    joint_sse = jnp.sum(joint_parts)
    denom = float(_N * _SPLIT)
    loss = (0.1 * st[0] + st[1] + 0.1 * joint_sse) / denom + 0.01 * reg[0, 0]
    return hardCodes, loss


# speedup vs baseline: 1.0323x; 1.0323x over previous
import functools

import jax
import jax.numpy as jnp
from jax import lax
from jax.experimental import pallas as pl
from jax.experimental.pallas import tpu as pltpu
from jax.experimental.pallas import tpu_sc as plsc

_M = 4
_K = 8192
_D = 256
_SPLIT = _D // _M
_N = 8192
_TN = 1024

_NW = 32
_ROWS_W = (_M * _N) // _NW
_BLK = 128


def _main_body(x_ref, r_ref, c_ref, idx_ref, soft_ref, stats_ref):
    xt = x_ref[...]
    rm = r_ref[0]
    split = jnp.dot(xt, rm, preferred_element_type=jnp.float32)
    c = c_ref[0]
    cc = jnp.sum(c * c, axis=1)
    dots2 = jax.lax.dot_general(2.0 * split, c, (((1,), (1,)), ((), ())),
                                preferred_element_type=jnp.float32)
    s = dots2 - cc[None, :]
    mx = jnp.max(s, axis=1, keepdims=True)
    ge = s >= mx
    iota = jax.lax.broadcasted_iota(jnp.int32, (_TN, _K), 1)
    idx = jnp.min(jnp.where(ge, iota, _K), axis=1, keepdims=True)
    p = jnp.exp(s - mx).astype(jnp.bfloat16)
    lane65 = jax.lax.broadcasted_iota(jnp.int32, (_K, _SPLIT + 1), 1)
    c_ext = jnp.where(lane65 < _SPLIT, jnp.pad(c, ((0, 0), (0, 1))), 1.0)
    soft_den = jax.lax.dot_general(p, c_ext, (((1,), (0,)), ((), ())),
                                   preferred_element_type=jnp.float32)
    den = soft_den[:, _SPLIT:_SPLIT + 1]
    soft = soft_den[:, :_SPLIT] / den
    xx = jnp.sum(split * split, axis=1, keepdims=True)
    d1 = split - soft
    lane = jax.lax.broadcasted_iota(jnp.int32, (1, 128), 1)
    v = jnp.where(lane == 0, jnp.sum(d1 * d1),
                  jnp.where(lane == 1, jnp.sum(xx - mx), 0.0))
    stats_ref[...] = v.reshape(1, 1, 1, 128)
    soft_ref[...] = soft.reshape(1, _TN, _SPLIT)
    idx_ref[...] = idx.reshape(1, _TN, 1)


def _reg_body(r_ref, o_ref):
    r = r_ref[...]
    rrt = jax.lax.dot_general(r, r, (((1,), (1,)), ((), ())),
                              preferred_element_type=jnp.float32)
    i0 = jax.lax.broadcasted_iota(jnp.int32, (_D, _D), 0)
    i1 = jax.lax.broadcasted_iota(jnp.int32, (_D, _D), 1)
    d = rrt - (i0 == i1).astype(jnp.float32)
    o_ref[...] = (jnp.sum(d * d) / float(_D * _D)).reshape(1, 1)


def _joint_body(cs_hbm, idx_hbm, soft_hbm, out_hbm, idx_v, rows_v, soft_v,
                acc_v, sem):
    wid = lax.axis_index("s") * 2 + lax.axis_index("c")
    moff = (wid // (_N // _ROWS_W)) * _K

    zero = jnp.zeros((16,), jnp.float32)

    def blk_body(b, accs):
        base = wid * _ROWS_W + b * _BLK
        pltpu.sync_copy(idx_hbm.at[pl.ds(base, _BLK)], idx_v)
        for j in range(_BLK // 16):
            sl = pl.ds(j * 16, 16)
            idx_v[sl] = idx_v[sl] + moff
        pltpu.async_copy(cs_hbm.at[idx_v], rows_v, sem).wait()
        pltpu.sync_copy(soft_hbm.at[pl.ds(base, _BLK)], soft_v)

        def row_body(i, a):
            out = list(a)
            for rr in range(4):
                r = i * 4 + rr
                for g in range(4):
                    sl = pl.ds(g * 16, 16)
                    d = soft_v[r, sl] - rows_v[r, sl]
                    out[g] = out[g] + d * d
            return tuple(out)

        return lax.fori_loop(0, _BLK // 4, row_body, accs)

    accs = lax.fori_loop(0, _ROWS_W // _BLK, blk_body,
                         (zero, zero, zero, zero))
    acc_v[pl.ds(0, 16)] = accs[0] + accs[1] + accs[2] + accs[3]
    pltpu.sync_copy(acc_v.at[pl.ds(0, 16)], out_hbm.at[pl.ds(wid * 16, 16)])


def kernel(x, codebook0, codebook1, codebook2, codebook3, rotateMatrix):
    cs = jnp.stack([codebook0, codebook1, codebook2, codebook3], axis=0)
    rs = rotateMatrix.reshape(_D, _M, _SPLIT).transpose(1, 0, 2)
    nt = _N // _TN
    idx_out, soft, stats = pl.pallas_call(
        _main_body,
        grid=(_M, nt),
        in_specs=[
            pl.BlockSpec((_TN, _D), lambda m, n: (n, 0)),
            pl.BlockSpec((1, _D, _SPLIT), lambda m, n: (m, 0, 0)),
            pl.BlockSpec((1, _K, _SPLIT), lambda m, n: (m, 0, 0)),
        ],
        out_specs=[
            pl.BlockSpec((1, _TN, 1), lambda m, n: (m, n, 0)),
            pl.BlockSpec((1, _TN, _SPLIT), lambda m, n: (m, n, 0)),
            pl.BlockSpec((1, 1, 1, 128), lambda m, n: (m, n, 0, 0)),
        ],
        out_shape=[
            jax.ShapeDtypeStruct((_M, _N, 1), jnp.int32),
            jax.ShapeDtypeStruct((_M, _N, _SPLIT), jnp.float32),
            jax.ShapeDtypeStruct((_M, nt, 1, 128), jnp.float32),
        ],
    )(x, rs, cs)
    reg = pl.pallas_call(
        _reg_body,
        out_shape=jax.ShapeDtypeStruct((1, 1), jnp.float32),
    )(rotateMatrix)

    cs_pad = jnp.pad(cs.reshape(_M * _K, _SPLIT), ((0, 0), (0, 128 - _SPLIT)))
    joint_parts = pl.kernel(
        _joint_body,
        out_type=jax.ShapeDtypeStruct((_NW * 16,), jnp.float32),
        mesh=plsc.VectorSubcoreMesh(core_axis_name="c", subcore_axis_name="s"),
        scratch_types=[
            pltpu.VMEM((_BLK,), jnp.int32),
            pltpu.VMEM((_BLK, 128), jnp.float32),
            pltpu.VMEM((_BLK, _SPLIT), jnp.float32),
            pltpu.VMEM((64,), jnp.float32),
            pltpu.SemaphoreType.DMA,
        ],
    )(cs_pad, idx_out.reshape(_M * _N), soft.reshape(_M * _N, _SPLIT))

    hardCodes = idx_out.reshape(_M, _N).T
    st = jnp.sum(stats.reshape(_M * nt, 128), axis=0)
    joint_sse = jnp.sum(joint_parts)
    denom = float(_N * _SPLIT)
    loss = (0.1 * st[0] + st[1] + 0.1 * joint_sse) / denom + 0.01 * reg[0, 0]
    return hardCodes, loss


# speedup vs baseline: 1.3238x; 1.2824x over previous
import jax
import jax.numpy as jnp
from jax.experimental import pallas as pl

_M = 4
_K = 8192
_D = 256
_SPLIT = _D // _M
_N = 8192
_TN = 1024


def _main_body(x_ref, r_ref, c_ref, idx_ref, stats_ref):
    xt = x_ref[...]
    rm = r_ref[0]
    split = jnp.dot(xt, rm, preferred_element_type=jnp.float32)
    c = c_ref[0]
    cc = jnp.sum(c * c, axis=1)
    dots2 = jax.lax.dot_general(2.0 * split, c, (((1,), (1,)), ((), ())),
                                preferred_element_type=jnp.float32)
    s = dots2 - cc[None, :]
    mx = jnp.max(s, axis=1, keepdims=True)
    ge = s >= mx
    iota = jax.lax.broadcasted_iota(jnp.int32, (_TN, _K), 1)
    idx = jnp.min(jnp.where(ge, iota, _K), axis=1, keepdims=True)
    p = jnp.exp(s - mx).astype(jnp.bfloat16)
    oh = jnp.where(ge, 1.0, 0.0)
    lane65 = jax.lax.broadcasted_iota(jnp.int32, (_K, _SPLIT + 1), 1)
    c_ext = jnp.where(lane65 < _SPLIT, jnp.pad(c, ((0, 0), (0, 1))), 1.0)
    soft_den = jax.lax.dot_general(p, c_ext, (((1,), (0,)), ((), ())),
                                   preferred_element_type=jnp.float32)
    den = soft_den[:, _SPLIT:_SPLIT + 1]
    soft = soft_den[:, :_SPLIT] / den
    hard = jax.lax.dot_general(oh, c, (((1,), (0,)), ((), ())),
                               preferred_element_type=jnp.float32)
    d1 = split - soft
    d2 = split - hard
    d3 = soft - hard
    lane = jax.lax.broadcasted_iota(jnp.int32, (1, 128), 1)
    v = jnp.where(lane == 0, jnp.sum(d1 * d1),
                  jnp.where(lane == 1, jnp.sum(d2 * d2),
                            jnp.where(lane == 2, jnp.sum(d3 * d3), 0.0)))
    stats_ref[...] = v.reshape(1, 1, 1, 128)
    idx_ref[...] = idx.reshape(1, _TN, 1)


def _reg_body(r_ref, o_ref):
    r = r_ref[...]
    rrt = jax.lax.dot_general(r, r, (((1,), (1,)), ((), ())),
                              preferred_element_type=jnp.float32)
    i0 = jax.lax.broadcasted_iota(jnp.int32, (_D, _D), 0)
    i1 = jax.lax.broadcasted_iota(jnp.int32, (_D, _D), 1)
    d = rrt - (i0 == i1).astype(jnp.float32)
    o_ref[...] = (jnp.sum(d * d) / float(_D * _D)).reshape(1, 1)


def kernel(x, codebook0, codebook1, codebook2, codebook3, rotateMatrix):
    cs = jnp.stack([codebook0, codebook1, codebook2, codebook3], axis=0)
    rs = rotateMatrix.reshape(_D, _M, _SPLIT).transpose(1, 0, 2)
    nt = _N // _TN
    idx_out, stats = pl.pallas_call(
        _main_body,
        grid=(_M, nt),
        in_specs=[
            pl.BlockSpec((_TN, _D), lambda m, n: (n, 0)),
            pl.BlockSpec((1, _D, _SPLIT), lambda m, n: (m, 0, 0)),
            pl.BlockSpec((1, _K, _SPLIT), lambda m, n: (m, 0, 0)),
        ],
        out_specs=[
            pl.BlockSpec((1, _TN, 1), lambda m, n: (m, n, 0)),
            pl.BlockSpec((1, 1, 1, 128), lambda m, n: (m, n, 0, 0)),
        ],
        out_shape=[
            jax.ShapeDtypeStruct((_M, _N, 1), jnp.int32),
            jax.ShapeDtypeStruct((_M, nt, 1, 128), jnp.float32),
        ],
    )(x, rs, cs)
    reg = pl.pallas_call(
        _reg_body,
        out_shape=jax.ShapeDtypeStruct((1, 1), jnp.float32),
    )(rotateMatrix)
    hardCodes = idx_out.reshape(_M, _N).T
    s = jnp.sum(stats.reshape(_M * nt, 128), axis=0)
    denom = float(_N * _SPLIT)
    loss = (0.1 * s[0] + s[1] + 0.1 * s[2]) / denom + 0.01 * reg[0, 0]
    return hardCodes, loss


# speedup vs baseline: 1.5070x; 1.1384x over previous
import jax
import jax.numpy as jnp
from jax.experimental import pallas as pl

_M = 4
_K = 8192
_D = 256
_SPLIT = _D // _M
_N = 8192
_TN = 1024


def _main_body(x_ref, r_ref, c_ref, idx_ref, stats_ref):
    xt = x_ref[...]
    rm = r_ref[0]
    split = jnp.dot(xt, rm, preferred_element_type=jnp.float32)
    c = c_ref[0]
    cc = jnp.sum(c * c, axis=1)
    dots2 = jax.lax.dot_general(2.0 * split, c, (((1,), (1,)), ((), ())),
                                preferred_element_type=jnp.float32)
    s = dots2 - cc[None, :]
    mx = jnp.max(s, axis=1, keepdims=True)
    ge = s >= mx
    idx = jnp.argmax(s, axis=1).astype(jnp.int32).reshape(_TN, 1)
    p = jnp.exp(s - mx).astype(jnp.bfloat16)
    oh = jnp.where(ge, 1.0, 0.0)
    lane65 = jax.lax.broadcasted_iota(jnp.int32, (_K, _SPLIT + 1), 1)
    c_ext = jnp.where(lane65 < _SPLIT, jnp.pad(c, ((0, 0), (0, 1))), 1.0)
    soft_den = jax.lax.dot_general(p, c_ext, (((1,), (0,)), ((), ())),
                                   preferred_element_type=jnp.float32)
    den = soft_den[:, _SPLIT:_SPLIT + 1]
    soft = soft_den[:, :_SPLIT] / den
    hard = jax.lax.dot_general(oh, c, (((1,), (0,)), ((), ())),
                               preferred_element_type=jnp.float32)
    d1 = split - soft
    d2 = split - hard
    d3 = soft - hard
    lane = jax.lax.broadcasted_iota(jnp.int32, (1, 128), 1)
    v = jnp.where(lane == 0, jnp.sum(d1 * d1),
                  jnp.where(lane == 1, jnp.sum(d2 * d2),
                            jnp.where(lane == 2, jnp.sum(d3 * d3), 0.0)))
    stats_ref[...] = v.reshape(1, 1, 1, 128)
    idx_ref[...] = idx.reshape(1, _TN, 1)


def _reg_body(r_ref, o_ref):
    r = r_ref[...]
    rrt = jax.lax.dot_general(r, r, (((1,), (1,)), ((), ())),
                              preferred_element_type=jnp.float32)
    i0 = jax.lax.broadcasted_iota(jnp.int32, (_D, _D), 0)
    i1 = jax.lax.broadcasted_iota(jnp.int32, (_D, _D), 1)
    d = rrt - (i0 == i1).astype(jnp.float32)
    o_ref[...] = (jnp.sum(d * d) / float(_D * _D)).reshape(1, 1)


def kernel(x, codebook0, codebook1, codebook2, codebook3, rotateMatrix):
    cs = jnp.stack([codebook0, codebook1, codebook2, codebook3], axis=0)
    rs = rotateMatrix.reshape(_D, _M, _SPLIT).transpose(1, 0, 2)
    nt = _N // _TN
    idx_out, stats = pl.pallas_call(
        _main_body,
        grid=(_M, nt),
        in_specs=[
            pl.BlockSpec((_TN, _D), lambda m, n: (n, 0)),
            pl.BlockSpec((1, _D, _SPLIT), lambda m, n: (m, 0, 0)),
            pl.BlockSpec((1, _K, _SPLIT), lambda m, n: (m, 0, 0)),
        ],
        out_specs=[
            pl.BlockSpec((1, _TN, 1), lambda m, n: (m, n, 0)),
            pl.BlockSpec((1, 1, 1, 128), lambda m, n: (m, n, 0, 0)),
        ],
        out_shape=[
            jax.ShapeDtypeStruct((_M, _N, 1), jnp.int32),
            jax.ShapeDtypeStruct((_M, nt, 1, 128), jnp.float32),
        ],
    )(x, rs, cs)
    reg = pl.pallas_call(
        _reg_body,
        out_shape=jax.ShapeDtypeStruct((1, 1), jnp.float32),
    )(rotateMatrix)
    hardCodes = idx_out.reshape(_M, _N).T
    s = jnp.sum(stats.reshape(_M * nt, 128), axis=0)
    denom = float(_N * _SPLIT)
    loss = (0.1 * s[0] + s[1] + 0.1 * s[2]) / denom + 0.01 * reg[0, 0]
    return hardCodes, loss
